# baseline (speedup 1.0000x reference)
import jax
import jax.numpy as jnp
from jax.experimental import pallas as pl
from jax.experimental.pallas import tpu as pltpu

NUM_TOKENS = 32768
MODEL_DIM = 1024
NUM_EXPERTS = 64
BLOCK_T = 512
NUM_BLOCKS = NUM_TOKENS // BLOCK_T


def _gate_body(x_ref, w_ref, utri_ref, eidx_ref,
               idx_ref, loc_ref, gate_ref, laux_ref, me_acc, cnt_acc):
    i = pl.program_id(0)

    @pl.when(i == 0)
    def _init():
        me_acc[...] = jnp.zeros_like(me_acc)
        cnt_acc[...] = jnp.zeros_like(cnt_acc)

    E, B = NUM_EXPERTS, BLOCK_T
    lg = jax.lax.dot_general(
        w_ref[...], x_ref[...], (((1,), (1,)), ((), ())),
        preferred_element_type=jnp.float32)

    eidx_f = eidx_ref[...]
    rowmax = jnp.max(lg, axis=0, keepdims=True)
    is_max = lg == rowmax
    idx_f = jnp.min(jnp.where(is_max, eidx_f, float(E)),
                    axis=0, keepdims=True)

    exps = jnp.exp(lg - rowmax)
    denom = jnp.sum(exps, axis=0, keepdims=True)
    gate = 1.0 / denom
    mask = (eidx_f == idx_f).astype(jnp.float32)

    csum = jax.lax.dot_general(
        mask.astype(jnp.bfloat16), utri_ref[...], (((1,), (0,)), ((), ())),
        preferred_element_type=jnp.float32)
    loc_local = jnp.sum(csum * mask, axis=0, keepdims=True)
    loc_carry = jax.lax.dot_general(
        cnt_acc[...], mask, (((1,), (0,)), ((), ())),
        preferred_element_type=jnp.float32)

    ones_row = jnp.ones((1, B), jnp.float32)
    me_part = jax.lax.dot_general(
        gate, exps, (((1,), (1,)), ((), ())),
        preferred_element_type=jnp.float32)
    ce_part = jax.lax.dot_general(
        ones_row, mask, (((1,), (1,)), ((), ())),
        preferred_element_type=jnp.float32)

    idx_ref[...] = idx_f.astype(jnp.int32).reshape(1, 1, B)
    loc_ref[...] = (loc_local + loc_carry).astype(jnp.int32).reshape(1, 1, B)
    gate_ref[...] = gate.reshape(1, 1, B)
    me_acc[...] += me_part
    cnt_acc[...] += ce_part

    @pl.when(i == NUM_BLOCKS - 1)
    def _fin():
        laux_ref[0, 0] = (jnp.sum(me_acc[...] * cnt_acc[...])
                          * (NUM_EXPERTS / (NUM_TOKENS * NUM_TOKENS)))


def kernel(input, W):
    num_tokens, num_experts = NUM_TOKENS, NUM_EXPERTS
    capacity = int((num_tokens + num_experts - 1) // num_experts)
    B = BLOCK_T

    row_i = jax.ShapeDtypeStruct((NUM_BLOCKS, 1, B), jnp.int32)
    row_f = jax.ShapeDtypeStruct((NUM_BLOCKS, 1, B), jnp.float32)
    pallas_fn = pl.pallas_call(
        _gate_body,
        grid=(NUM_BLOCKS,),
        in_specs=[
            pl.BlockSpec((B, MODEL_DIM), lambda i: (i, 0)),
            pl.BlockSpec((NUM_EXPERTS, MODEL_DIM), lambda i: (0, 0)),
            pl.BlockSpec((B, B), lambda i: (0, 0)),
            pl.BlockSpec((NUM_EXPERTS, B), lambda i: (0, 0)),
        ],
        out_specs=[
            pl.BlockSpec((1, 1, B), lambda i: (i, 0, 0)),
            pl.BlockSpec((1, 1, B), lambda i: (i, 0, 0)),
            pl.BlockSpec((1, 1, B), lambda i: (i, 0, 0)),
            pl.BlockSpec(memory_space=pltpu.SMEM),
        ],
        out_shape=[
            row_i, row_i, row_f,
            jax.ShapeDtypeStruct((1, 1), jnp.float32),
        ],
        scratch_shapes=[
            pltpu.VMEM((1, NUM_EXPERTS), jnp.float32),
            pltpu.VMEM((1, NUM_EXPERTS), jnp.float32),
        ],
    )

    s = jax.lax.broadcasted_iota(jnp.int32, (B, B), 0)
    t = jax.lax.broadcasted_iota(jnp.int32, (B, B), 1)
    utri = (s < t).astype(jnp.bfloat16)
    eidx = jax.lax.broadcasted_iota(
        jnp.int32, (num_experts, B), 0).astype(jnp.float32)

    idx3, loc3, gate3, laux = pallas_fn(input, W, utri, eidx)
    return (laux[0, 0], idx3.reshape(num_tokens), capacity,
            loc3.reshape(num_tokens), gate3.reshape(num_tokens), num_experts)


# speedup vs baseline: 1.2276x; 1.2276x over previous
import jax
import jax.numpy as jnp
from jax.experimental import pallas as pl
from jax.experimental.pallas import tpu as pltpu

NUM_TOKENS = 32768
MODEL_DIM = 1024
NUM_EXPERTS = 64
BLOCK_T = 2048
NUM_BLOCKS = NUM_TOKENS // BLOCK_T


def _gate_body(x_ref, w_ref, utri_ref, eidx_ref,
               idx_ref, loc_ref, gate_ref, laux_ref, me_acc, cnt_acc):
    i = pl.program_id(0)

    @pl.when(i == 0)
    def _init():
        me_acc[...] = jnp.zeros_like(me_acc)
        cnt_acc[...] = jnp.zeros_like(cnt_acc)

    E, B = NUM_EXPERTS, BLOCK_T
    lg = jax.lax.dot_general(
        w_ref[...], x_ref[...], (((1,), (1,)), ((), ())),
        preferred_element_type=jnp.float32)

    eidx_f = eidx_ref[...]
    rowmax = jnp.max(lg, axis=0, keepdims=True)
    is_max = lg == rowmax
    idx_f = jnp.min(jnp.where(is_max, eidx_f, float(E)),
                    axis=0, keepdims=True)

    exps = jnp.exp(lg - rowmax)
    denom = jnp.sum(exps, axis=0, keepdims=True)
    gate = 1.0 / denom
    mask = (eidx_f == idx_f).astype(jnp.float32)

    csum = jax.lax.dot_general(
        mask.astype(jnp.bfloat16), utri_ref[...], (((1,), (0,)), ((), ())),
        preferred_element_type=jnp.float32)
    loc_local = jnp.sum(csum * mask, axis=0, keepdims=True)
    loc_carry = jax.lax.dot_general(
        cnt_acc[...], mask, (((1,), (0,)), ((), ())),
        preferred_element_type=jnp.float32)

    ones_row = jnp.ones((1, B), jnp.float32)
    me_part = jax.lax.dot_general(
        gate, exps, (((1,), (1,)), ((), ())),
        preferred_element_type=jnp.float32)
    ce_part = jax.lax.dot_general(
        ones_row, mask, (((1,), (1,)), ((), ())),
        preferred_element_type=jnp.float32)

    idx_ref[...] = idx_f.astype(jnp.int32).reshape(1, 1, B)
    loc_ref[...] = (loc_local + loc_carry).astype(jnp.int32).reshape(1, 1, B)
    gate_ref[...] = gate.reshape(1, 1, B)
    me_acc[...] += me_part
    cnt_acc[...] += ce_part

    @pl.when(i == NUM_BLOCKS - 1)
    def _fin():
        laux_ref[0, 0] = (jnp.sum(me_acc[...] * cnt_acc[...])
                          * (NUM_EXPERTS / (NUM_TOKENS * NUM_TOKENS)))


def kernel(input, W):
    num_tokens, num_experts = NUM_TOKENS, NUM_EXPERTS
    capacity = int((num_tokens + num_experts - 1) // num_experts)
    B = BLOCK_T

    row_i = jax.ShapeDtypeStruct((NUM_BLOCKS, 1, B), jnp.int32)
    row_f = jax.ShapeDtypeStruct((NUM_BLOCKS, 1, B), jnp.float32)
    pallas_fn = pl.pallas_call(
        _gate_body,
        grid=(NUM_BLOCKS,),
        in_specs=[
            pl.BlockSpec((B, MODEL_DIM), lambda i: (i, 0)),
            pl.BlockSpec((NUM_EXPERTS, MODEL_DIM), lambda i: (0, 0)),
            pl.BlockSpec((B, B), lambda i: (0, 0)),
            pl.BlockSpec((NUM_EXPERTS, B), lambda i: (0, 0)),
        ],
        out_specs=[
            pl.BlockSpec((1, 1, B), lambda i: (i, 0, 0)),
            pl.BlockSpec((1, 1, B), lambda i: (i, 0, 0)),
            pl.BlockSpec((1, 1, B), lambda i: (i, 0, 0)),
            pl.BlockSpec(memory_space=pltpu.SMEM),
        ],
        out_shape=[
            row_i, row_i, row_f,
            jax.ShapeDtypeStruct((1, 1), jnp.float32),
        ],
        scratch_shapes=[
            pltpu.VMEM((1, NUM_EXPERTS), jnp.float32),
            pltpu.VMEM((1, NUM_EXPERTS), jnp.float32),
        ],
    )

    s = jax.lax.broadcasted_iota(jnp.int32, (B, B), 0)
    t = jax.lax.broadcasted_iota(jnp.int32, (B, B), 1)
    utri = (s < t).astype(jnp.bfloat16)
    eidx = jax.lax.broadcasted_iota(
        jnp.int32, (num_experts, B), 0).astype(jnp.float32)

    idx3, loc3, gate3, laux = pallas_fn(input, W, utri, eidx)
    return (laux[0, 0], idx3.reshape(num_tokens), capacity,
            loc3.reshape(num_tokens), gate3.reshape(num_tokens), num_experts)


# speedup vs baseline: 1.3167x; 1.0726x over previous
import jax
import jax.numpy as jnp
from jax.experimental import pallas as pl
from jax.experimental.pallas import tpu as pltpu

NUM_TOKENS = 32768
MODEL_DIM = 1024
NUM_EXPERTS = 64
BLOCK_T = 1024
NUM_BLOCKS = NUM_TOKENS // BLOCK_T
NUM_STEPS = NUM_BLOCKS // 2


def _half(x, w, utri, eidx_f, cnt_row):
    E, B = NUM_EXPERTS, BLOCK_T
    lg = jax.lax.dot_general(
        w, x, (((1,), (1,)), ((), ())),
        preferred_element_type=jnp.float32)

    rowmax = jnp.max(lg, axis=0, keepdims=True)
    is_max = lg == rowmax
    idx_f = jnp.min(jnp.where(is_max, eidx_f, float(E)),
                    axis=0, keepdims=True)

    exps = jnp.exp(lg - rowmax)
    denom = jnp.sum(exps, axis=0, keepdims=True)
    gate = 1.0 / denom
    mask = (eidx_f == idx_f).astype(jnp.float32)

    csum = jax.lax.dot_general(
        mask.astype(jnp.bfloat16), utri, (((1,), (0,)), ((), ())),
        preferred_element_type=jnp.float32)
    loc_local = jnp.sum(csum * mask, axis=0, keepdims=True)
    loc_carry = jax.lax.dot_general(
        cnt_row, mask, (((1,), (0,)), ((), ())),
        preferred_element_type=jnp.float32)
    loc = loc_local + loc_carry

    ones_row = jnp.ones((1, B), jnp.float32)
    me_part = jax.lax.dot_general(
        gate, exps, (((1,), (1,)), ((), ())),
        preferred_element_type=jnp.float32)
    ce_part = jax.lax.dot_general(
        ones_row, mask, (((1,), (1,)), ((), ())),
        preferred_element_type=jnp.float32)
    return idx_f, loc, gate, me_part, ce_part


def _gate_body(x0_ref, x1_ref, w_ref, utri_ref, eidx_ref,
               idx_ref, loc_ref, gate_ref, laux_ref, me_acc, cnt_acc):
    i = pl.program_id(0)

    @pl.when(i == 0)
    def _init():
        me_acc[...] = jnp.zeros_like(me_acc)
        cnt_acc[...] = jnp.zeros_like(cnt_acc)

    B = BLOCK_T
    w = w_ref[...]
    utri = utri_ref[...]
    eidx_f = eidx_ref[...]
    cnt0 = cnt_acc[...]

    idx0, loc0, gate0, me0, ce0 = _half(x0_ref[...], w, utri, eidx_f, cnt0)
    idx1, loc1, gate1, me1, ce1 = _half(x1_ref[...], w, utri, eidx_f,
                                        cnt0 + ce0)

    idx_ref[0, 0, :] = idx0.astype(jnp.int32).reshape(B)
    idx_ref[0, 1, :] = idx1.astype(jnp.int32).reshape(B)
    loc_ref[0, 0, :] = loc0.astype(jnp.int32).reshape(B)
    loc_ref[0, 1, :] = loc1.astype(jnp.int32).reshape(B)
    gate_ref[0, 0, :] = gate0.reshape(B)
    gate_ref[0, 1, :] = gate1.reshape(B)
    me_acc[...] += me0 + me1
    cnt_acc[...] += ce0 + ce1

    @pl.when(i == NUM_STEPS - 1)
    def _fin():
        laux_ref[0, 0] = (jnp.sum(me_acc[...] * cnt_acc[...])
                          * (NUM_EXPERTS / (NUM_TOKENS * NUM_TOKENS)))


def kernel(input, W):
    num_tokens, num_experts = NUM_TOKENS, NUM_EXPERTS
    capacity = int((num_tokens + num_experts - 1) // num_experts)
    B = BLOCK_T

    row_i = jax.ShapeDtypeStruct((NUM_STEPS, 2, B), jnp.int32)
    row_f = jax.ShapeDtypeStruct((NUM_STEPS, 2, B), jnp.float32)
    pallas_fn = pl.pallas_call(
        _gate_body,
        grid=(NUM_STEPS,),
        in_specs=[
            pl.BlockSpec((B, MODEL_DIM), lambda i: (2 * i, 0)),
            pl.BlockSpec((B, MODEL_DIM), lambda i: (2 * i + 1, 0)),
            pl.BlockSpec((NUM_EXPERTS, MODEL_DIM), lambda i: (0, 0)),
            pl.BlockSpec((B, B), lambda i: (0, 0)),
            pl.BlockSpec((NUM_EXPERTS, B), lambda i: (0, 0)),
        ],
        out_specs=[
            pl.BlockSpec((1, 2, B), lambda i: (i, 0, 0)),
            pl.BlockSpec((1, 2, B), lambda i: (i, 0, 0)),
            pl.BlockSpec((1, 2, B), lambda i: (i, 0, 0)),
            pl.BlockSpec(memory_space=pltpu.SMEM),
        ],
        out_shape=[
            row_i, row_i, row_f,
            jax.ShapeDtypeStruct((1, 1), jnp.float32),
        ],
        scratch_shapes=[
            pltpu.VMEM((1, NUM_EXPERTS), jnp.float32),
            pltpu.VMEM((1, NUM_EXPERTS), jnp.float32),
        ],
    )

    s = jax.lax.broadcasted_iota(jnp.int32, (B, B), 0)
    t = jax.lax.broadcasted_iota(jnp.int32, (B, B), 1)
    utri = (s < t).astype(jnp.bfloat16)
    eidx = jax.lax.broadcasted_iota(
        jnp.int32, (num_experts, B), 0).astype(jnp.float32)

    idx3, loc3, gate3, laux = pallas_fn(input, input, W, utri, eidx)
    return (laux[0, 0], idx3.reshape(num_tokens), capacity,
            loc3.reshape(num_tokens), gate3.reshape(num_tokens), num_experts)
